# Initial kernel scaffold; baseline (speedup 1.0000x reference)
import jax
import jax.numpy as jnp
from jax.experimental import pallas as pl


def kernel(x, A, p, W1, b1, W2, b2, W3, b3, W_lin, b_lin):
    raise NotImplementedError("write your pallas kernel here")


# speedup vs baseline: 80.8046x; 80.8046x over previous
import functools

import jax
import jax.numpy as jnp
from jax import lax
from jax.experimental import pallas as pl
from jax.experimental.pallas import tpu as pltpu
from jax.experimental.pallas import tpu_sc as plsc

N = 1024
E = 65536
NSUB = 16
EPW = E // NSUB
ROWS = EPW // 128
NV = N // 16


def _dense_body(x_ref, w1_ref, b1_ref, w2_ref, b2_ref, w3_ref, b3_ref,
                blin_ref, u0_ref, scal_ref):
    w23 = jnp.dot(w2_ref[...], w3_ref[...], preferred_element_type=jnp.float32)
    v = jnp.dot(w1_ref[...], w23, preferred_element_type=jnp.float32)
    u0_ref[...] = jnp.dot(x_ref[...], v, preferred_element_type=jnp.float32)
    s1 = jnp.dot(b1_ref[...], w23, preferred_element_type=jnp.float32)[0, 0]
    s2 = jnp.dot(b2_ref[...], w3_ref[...], preferred_element_type=jnp.float32)[0, 0]
    lane = lax.broadcasted_iota(jnp.int32, (8, 128), 1)
    rowi = lax.broadcasted_iota(jnp.int32, (8, 128), 0)
    r0 = rowi == 0
    scal = (jnp.where(r0 & (lane == 0), s1, 0.0)
            + jnp.where(r0 & (lane == 1), s2, 0.0)
            + jnp.where(r0 & (lane == 2), b3_ref[0, 0], 0.0)
            + jnp.where(r0 & (lane == 3), blin_ref[0, 0], 0.0))
    scal_ref[...] = scal


def _dense_tc(x, W1, b1, W2, b2, W3, b3, b_lin):
    return pl.pallas_call(
        _dense_body,
        out_shape=(
            jax.ShapeDtypeStruct((N, 1), jnp.float32),
            jax.ShapeDtypeStruct((8, 128), jnp.float32),
        ),
    )(x, W1, b1.reshape(1, N), W2, b2.reshape(1, N), W3,
      b3.reshape(1, 1), b_lin.reshape(1, 1))


def _rsqrt16(d):
    i = lax.bitcast_convert_type(d, jnp.int32)
    y = lax.bitcast_convert_type(jnp.int32(0x5F3759DF) - (i >> 1), jnp.float32)
    for _ in range(4):
        y = y * (1.5 - 0.5 * d * y * y)
    return y


def _gcn_body(row_h, col_h, p_h, u0_h, wlin_h, scal_h, out_h,
              row_v, col_v, nrm_v, msg_t, msg_r,
              t_tab, r_tab, r1_tab, dinv_t, tmp_t, tmp_r,
              wlin_tab, scal_tab, ones64, zero64, out_v, sem,
              sh_ta, sh_ra, sh_tb, sh_rb):
    wid = lax.axis_index("s")
    chunk = pl.ds(wid * (N // NSUB), N // NSUB)

    pltpu.async_copy(row_h.at[wid], row_v, sem)
    pltpu.async_copy(col_h.at[wid], col_v, sem)
    pltpu.async_copy(p_h.at[wid], nrm_v, sem)
    pltpu.async_copy(u0_h, t_tab, sem)
    pltpu.async_copy(scal_h, scal_tab, sem)

    z16 = jnp.zeros((16,), jnp.float32)
    o16 = jnp.ones((16,), jnp.float32)
    for k in range(4):
        ones64[pl.ds(16 * k, 16)] = o16
        zero64[pl.ds(16 * k, 16)] = z16

    def fill_r(k, _):
        r_tab[pl.ds(k * 16, 16)] = o16
        return 0
    lax.fori_loop(0, NV, fill_r, 0)

    pltpu.make_async_copy(row_h.at[wid], row_v, sem).wait()
    pltpu.make_async_copy(col_h.at[wid], col_v, sem).wait()
    pltpu.make_async_copy(p_h.at[wid], nrm_v, sem).wait()
    pltpu.make_async_copy(u0_h, t_tab, sem).wait()
    pltpu.make_async_copy(scal_h, scal_tab, sem).wait()

    pltpu.sync_copy(ones64, sh_ta.at[chunk])
    pltpu.sync_copy(zero64, sh_tb.at[chunk])
    pltpu.sync_copy(zero64, sh_rb.at[chunk])
    plsc.subcore_barrier()

    def deg_start(j, _):
        pltpu.async_copy(nrm_v.at[j], sh_ta.at[col_v.at[j]], sem, add=True)
        return 0
    lax.fori_loop(0, ROWS, deg_start, 0)

    def deg_wait(j, _):
        pltpu.make_async_copy(nrm_v.at[j], sh_ta.at[col_v.at[j]], sem).wait()
        return 0
    lax.fori_loop(0, ROWS, deg_wait, 0)
    plsc.subcore_barrier()

    pltpu.sync_copy(sh_ta, tmp_t)

    def mk_dinv(k, _):
        sl = pl.ds(k * 16, 16)
        dinv_t[sl] = _rsqrt16(tmp_t[sl])
        return 0
    lax.fori_loop(0, NV, mk_dinv, 0)

    def spmv(first, do_r, sh_t, sh_r, sh_nt, sh_nr):
        def epass(j, _):
            for k in range(8):
                sl = pl.ds(k * 16, 16)
                er = row_v[j, sl]
                if first:
                    dr = plsc.load_gather(dinv_t, [er])
                    dc = plsc.load_gather(dinv_t, [col_v[j, sl]])
                    nv = nrm_v[j, sl] * dr * dc
                    nrm_v[j, sl] = nv
                    msg_r[j, sl] = nv
                else:
                    nv = nrm_v[j, sl]
                    if do_r:
                        msg_r[j, sl] = nv * plsc.load_gather(r_tab, [er])
                msg_t[j, sl] = nv * plsc.load_gather(t_tab, [er])
            pltpu.async_copy(msg_t.at[j], sh_t.at[col_v.at[j]], sem, add=True)
            if do_r:
                pltpu.async_copy(msg_r.at[j], sh_r.at[col_v.at[j]], sem, add=True)
            return 0
        lax.fori_loop(0, ROWS, epass, 0)

        def edrain(j, _):
            pltpu.make_async_copy(msg_t.at[j], sh_t.at[col_v.at[j]], sem).wait()
            if do_r:
                pltpu.make_async_copy(msg_r.at[j], sh_r.at[col_v.at[j]], sem).wait()
            return 0
        lax.fori_loop(0, ROWS, edrain, 0)
        plsc.subcore_barrier()

        pltpu.sync_copy(sh_t, tmp_t)
        if do_r:
            pltpu.sync_copy(sh_r, tmp_r)
        if sh_nt is not None:
            pltpu.sync_copy(zero64, sh_nt.at[chunk])
            pltpu.sync_copy(zero64, sh_nr.at[chunk])

        def readback(k, _):
            sl = pl.ds(k * 16, 16)
            dv = dinv_t[sl]
            dd = dv * dv
            t_tab[sl] = tmp_t[sl] + dd * t_tab[sl]
            if do_r:
                r_tab[sl] = tmp_r[sl] + dd * r_tab[sl]
            return 0
        lax.fori_loop(0, NV, readback, 0)
        if sh_nt is not None:
            plsc.subcore_barrier()

    spmv(True, True, sh_tb, sh_rb, sh_ta, sh_ra)

    def save_r1(k, _):
        sl = pl.ds(k * 16, 16)
        r1_tab[sl] = r_tab[sl]
        return 0
    lax.fori_loop(0, NV, save_r1, 0)

    spmv(False, True, sh_ta, sh_ra, sh_tb, sh_rb)
    spmv(False, False, sh_tb, sh_rb, None, None)

    @pl.when(wid == 0)
    def _final():
        pltpu.sync_copy(wlin_h, wlin_tab)

        def _splat(i):
            return plsc.load_gather(scal_tab, [jnp.full((16,), i, jnp.int32)])
        s1v, s2v, b3v, blv = _splat(0), _splat(1), _splat(2), _splat(3)

        def mk_h3(k, m_acc):
            sl = pl.ds(k * 16, 16)
            h = t_tab[sl] + s1v * r_tab[sl] + s2v * r1_tab[sl] + b3v
            tmp_t[sl] = h
            return jnp.maximum(m_acc, h)
        m_acc = lax.fori_loop(0, NV, mk_h3, jnp.full((16,), -1e30, jnp.float32))
        m = jnp.max(m_acc)

        def mk_exp(k, carry):
            den_acc, num_acc = carry
            sl = pl.ds(k * 16, 16)
            e = jnp.exp(tmp_t[sl] - m)
            return den_acc + e, num_acc + e * wlin_tab[sl]
        den_acc, num_acc = lax.fori_loop(0, NV, mk_exp, (z16, z16))
        num_v = z16 + jnp.sum(num_acc)
        den_v = z16 + jnp.sum(den_acc)
        out_v[pl.ds(0, 16)] = num_v / den_v + blv
        pltpu.sync_copy(out_v, out_h)


_gcn_sc = functools.partial(
    pl.kernel,
    mesh=plsc.VectorSubcoreMesh(core_axis_name="c", subcore_axis_name="s",
                                num_cores=1),
    out_type=jax.ShapeDtypeStruct((16,), jnp.float32),
    compiler_params=pltpu.CompilerParams(needs_layout_passes=False),
    scratch_types=[
        pltpu.VMEM((ROWS, 128), jnp.int32),
        pltpu.VMEM((ROWS, 128), jnp.int32),
        pltpu.VMEM((ROWS, 128), jnp.float32),
        pltpu.VMEM((ROWS, 128), jnp.float32),
        pltpu.VMEM((ROWS, 128), jnp.float32),
        pltpu.VMEM((N,), jnp.float32),
        pltpu.VMEM((N,), jnp.float32),
        pltpu.VMEM((N,), jnp.float32),
        pltpu.VMEM((N,), jnp.float32),
        pltpu.VMEM((N,), jnp.float32),
        pltpu.VMEM((N,), jnp.float32),
        pltpu.VMEM((N,), jnp.float32),
        pltpu.VMEM((16,), jnp.float32),
        pltpu.VMEM((64,), jnp.float32),
        pltpu.VMEM((64,), jnp.float32),
        pltpu.VMEM((16,), jnp.float32),
        pltpu.SemaphoreType.DMA,
        pltpu.VMEM_SHARED((N,), jnp.float32),
        pltpu.VMEM_SHARED((N,), jnp.float32),
        pltpu.VMEM_SHARED((N,), jnp.float32),
        pltpu.VMEM_SHARED((N,), jnp.float32),
    ],
)(_gcn_body)


def kernel(x, A, p, W1, b1, W2, b2, W3, b3, W_lin, b_lin):
    u0, scal = _dense_tc(x, W1, b1, W2, b2, W3, b3, b_lin)
    row3 = A[0].reshape(NSUB, ROWS, 128)
    col3 = A[1].reshape(NSUB, ROWS, 128)
    p3 = p.reshape(NSUB, ROWS, 128)
    out16 = _gcn_sc(row3, col3, p3, u0.reshape(N), W_lin.reshape(N),
                    scal[0, :16])
    return out16[:1]
